# baseline (speedup 1.0000x reference)
import functools

import jax
import jax.numpy as jnp
from jax import lax
from jax.experimental import pallas as pl
from jax.experimental.pallas import tpu as pltpu
from jax.experimental.pallas import tpu_sc as plsc

_NUM_EXPERTS = 64
_TOP_K = 8
_BT = 1024
_IDX_MASK = _NUM_EXPERTS - 1

_N_TOK = 16384
_NW = 32
_TPW = _N_TOK // _NW
_LANES = 16


def _score_block(x_ref, w_ref, keys_ref):
    s = jax.lax.dot_general(
        x_ref[...], w_ref[...],
        dimension_numbers=(((1,), (1,)), ((), ())),
        preferred_element_type=jnp.float32,
    )
    m = jnp.max(s, axis=-1, keepdims=True)
    e = jnp.exp(s - m)
    p = e / jnp.sum(e, axis=-1, keepdims=True)
    bits = jax.lax.bitcast_convert_type(p, jnp.int32)
    iota = jax.lax.broadcasted_iota(jnp.int32, p.shape, 1)
    keys_ref[...] = (bits & ~_IDX_MASK) | (_IDX_MASK - iota)


def _tc_scores(x, weight):
    n_tokens, _ = x.shape
    grid = (n_tokens // _BT,)
    return pl.pallas_call(
        _score_block,
        grid=grid,
        in_specs=[
            pl.BlockSpec((_BT, x.shape[1]), lambda i: (i, 0)),
            pl.BlockSpec(weight.shape, lambda i: (0, 0)),
        ],
        out_specs=[
            pl.BlockSpec((_BT, _NUM_EXPERTS), lambda i: (i, 0)),
        ],
        out_shape=[
            jax.ShapeDtypeStruct((n_tokens, _NUM_EXPERTS), jnp.int32),
        ],
        compiler_params=pltpu.CompilerParams(
            dimension_semantics=("parallel",),
        ),
    )(x, weight)


@functools.partial(
    pl.kernel,
    out_type=[
        jax.ShapeDtypeStruct((_N_TOK * _TOP_K,), jnp.float32),
        jax.ShapeDtypeStruct((_N_TOK * _TOP_K,), jnp.int32),
    ],
    mesh=plsc.VectorSubcoreMesh(core_axis_name="c", subcore_axis_name="s"),
    scratch_types=[
        pltpu.VMEM((_TPW, _NUM_EXPERTS), jnp.int32),
        pltpu.VMEM((_TPW * _TOP_K,), jnp.int32),
        pltpu.VMEM((_TPW * _TOP_K,), jnp.float32),
        pltpu.VMEM((_TPW * _TOP_K,), jnp.int32),
    ],
)
def _sc_topk(keys_hbm, wout_hbm, iout_hbm,
             keys_v, kv, wv, iv):
    wid = lax.axis_index("s") * 2 + lax.axis_index("c")
    base = wid * _TPW
    pltpu.sync_copy(keys_hbm.at[pl.ds(base, _TPW)], keys_v)

    lane = lax.iota(jnp.int32, _LANES)
    rot = [(lane + sh) & (_LANES - 1) for sh in (8, 4, 2, 1)]

    def bcast_max(v):
        for r in rot:
            v = jnp.maximum(v, v.at[r].get(mode="promise_in_bounds"))
        return v

    def token_body(t, carry):
        a = [keys_v[2 * t, pl.ds(j * _LANES, _LANES)] for j in range(4)]
        b = [keys_v[2 * t + 1, pl.ds(j * _LANES, _LANES)] for j in range(4)]
        acc = jnp.zeros((_LANES,), jnp.int32)
        for k in range(_TOP_K):
            sa = bcast_max(jnp.maximum(jnp.maximum(a[0], a[1]),
                                       jnp.maximum(a[2], a[3])))
            sb = bcast_max(jnp.maximum(jnp.maximum(b[0], b[1]),
                                       jnp.maximum(b[2], b[3])))
            acc = jnp.where(lane == k, sa, acc)
            acc = jnp.where(lane == _TOP_K + k, sb, acc)
            a = [jnp.where(v == sa, -1, v) for v in a]
            b = [jnp.where(v == sb, -1, v) for v in b]
        kv[pl.ds(t * _LANES, _LANES)] = acc
        return carry

    lax.fori_loop(0, _TPW // 2, token_body, 0)

    def decode_body(j, carry):
        kvec = kv[pl.ds(j * _LANES, _LANES)]
        ivec = _IDX_MASK - (kvec & _IDX_MASK)
        vvec = jax.lax.bitcast_convert_type(kvec & ~_IDX_MASK, jnp.float32)
        wv[pl.ds(j * _LANES, _LANES)] = vvec
        iv[pl.ds(j * _LANES, _LANES)] = ivec
        return carry

    lax.fori_loop(0, _TPW * _TOP_K // _LANES, decode_body, 0)

    pltpu.sync_copy(wv, wout_hbm.at[pl.ds(base * _TOP_K, _TPW * _TOP_K)])
    pltpu.sync_copy(iv, iout_hbm.at[pl.ds(base * _TOP_K, _TPW * _TOP_K)])


@jax.jit
def kernel(x, weight):
    n_tokens, _ = x.shape
    (keys,) = _tc_scores(x, weight)
    wflat, iflat = _sc_topk(keys)
    return (wflat.reshape(n_tokens, _TOP_K),
            iflat.reshape(n_tokens, _TOP_K))


# speedup vs baseline: 1.6248x; 1.6248x over previous
import functools

import jax
import jax.numpy as jnp
from jax.experimental import pallas as pl
from jax.experimental.pallas import tpu as pltpu

_NUM_EXPERTS = 64
_TOP_K = 8
_BT = 1024
_IDX_MASK = _NUM_EXPERTS - 1


def _router_block(x_ref, w_ref, wout_ref, iout_ref):
    s = jax.lax.dot_general(
        x_ref[...], w_ref[...],
        dimension_numbers=(((1,), (1,)), ((), ())),
        preferred_element_type=jnp.float32,
    )
    st = s.T
    m = jnp.max(st, axis=0, keepdims=True)
    e = jnp.exp(st - m)
    rscale = 1.0 / jnp.sum(e, axis=0, keepdims=True)

    iota = jax.lax.broadcasted_iota(jnp.int32, e.shape, 0)
    vals = []
    idxs = []
    work = e
    for _ in range(_TOP_K):
        mx = jnp.max(work, axis=0, keepdims=True)
        idx = jnp.min(jnp.where(work == mx, iota, _NUM_EXPERTS),
                      axis=0, keepdims=True)
        vals.append(mx)
        idxs.append(idx)
        work = jnp.where(iota == idx, -1.0, work)
    iout_ref[...] = jnp.concatenate(idxs, axis=0).T
    wout_ref[...] = (jnp.concatenate(vals, axis=0) * rscale).T


@jax.jit
def kernel(x, weight):
    n_tokens, _ = x.shape
    grid = (n_tokens // _BT,)
    wout, iout = pl.pallas_call(
        _router_block,
        grid=grid,
        in_specs=[
            pl.BlockSpec((_BT, x.shape[1]), lambda i: (i, 0)),
            pl.BlockSpec(weight.shape, lambda i: (0, 0)),
        ],
        out_specs=[
            pl.BlockSpec((_BT, _TOP_K), lambda i: (i, 0)),
            pl.BlockSpec((_BT, _TOP_K), lambda i: (i, 0)),
        ],
        out_shape=[
            jax.ShapeDtypeStruct((n_tokens, _TOP_K), jnp.float32),
            jax.ShapeDtypeStruct((n_tokens, _TOP_K), jnp.int32),
        ],
        compiler_params=pltpu.CompilerParams(
            dimension_semantics=("parallel",),
        ),
    )(x, weight)
    return wout, iout
